# baseline (speedup 1.0000x reference)
import jax
import jax.numpy as jnp
from jax import lax
from jax.experimental import pallas as pl
from jax.experimental.pallas import tpu as pltpu
from jax.experimental.pallas import tpu_sc as plsc

_D = 512
_NTOK = 4 * 8192

_NW = 32
_TOK_W = _NTOK // _NW
_CHUNK = 256
_GRP = 16


def _sc_body(x_hbm, o_hbm, buf0, buf1, sem):
    wid = lax.axis_index("s") * 2 + lax.axis_index("c")
    base = wid * _TOK_W
    lanes = lax.iota(jnp.int32, 16)

    right_dma = None
    for c in range(_TOK_W // _CHUNK):
        row0 = base + c * _CHUNK
        rows_hbm = pl.ds(row0, _CHUNK)
        chunk = buf0 if c % 2 == 0 else buf1

        if right_dma is not None:
            right_dma.wait()
        right_dma = pltpu.async_copy(
            x_hbm.at[rows_hbm, pl.ds(128, 384)],
            o_hbm.at[rows_hbm, pl.ds(128, 384)], sem)

        pltpu.sync_copy(x_hbm.at[rows_hbm, pl.ds(0, 128)], chunk)

        def group(g, carry, chunk=chunk):
            rows = g * _GRP + lanes

            def gat(col):
                idx = jnp.full((16,), col, jnp.int32)
                return plsc.load_gather(chunk, [rows, idx])

            mark = gat(0) > 0.5
            shl = gat(1) > 0.5
            shr = jnp.logical_and(jnp.logical_not(shl), gat(2) > 0.5)
            active = jnp.logical_and(mark, jnp.logical_or(shl, shr))

            def first_set(base_col):
                acc = jnp.full((16,), 16, jnp.int32)
                for k in range(16):
                    hit = gat(base_col + k) > 0.5
                    acc = jnp.minimum(acc, jnp.where(hit, k, 16))
                return jnp.where(acc == 16, 0, acc)

            lo = first_set(16)
            hi = first_set(32)
            sa = first_set(48)

            value = lo + 16 * hi
            shl_res = jnp.bitwise_and(jnp.left_shift(value, sa), 255)
            shr_res = jnp.right_shift(value, sa)
            res = jnp.where(shl, shl_res, shr_res)
            res_lo = 96 + jnp.bitwise_and(res, 15)
            res_hi = 112 + jnp.right_shift(res, 4)

            two = jnp.full((16,), 2.0, jnp.float32)
            plsc.addupdate_scatter(chunk, [rows, res_lo], two, mask=active)
            plsc.addupdate_scatter(chunk, [rows, res_hi], two, mask=active)
            return carry

        lax.fori_loop(0, _CHUNK // _GRP, group, 0)
        pltpu.sync_copy(chunk, o_hbm.at[rows_hbm, pl.ds(0, 128)])
    right_dma.wait()


_sc_kernel = pl.kernel(
    _sc_body,
    out_type=jax.ShapeDtypeStruct((_NTOK, _D), jnp.float32),
    mesh=plsc.VectorSubcoreMesh(core_axis_name="c", subcore_axis_name="s"),
    scratch_types=[pltpu.VMEM((_CHUNK, 128), jnp.float32),
                   pltpu.VMEM((_CHUNK, 128), jnp.float32),
                   pltpu.SemaphoreType.DMA],
    compiler_params=pltpu.CompilerParams(needs_layout_passes=False),
)


def kernel(x_bd):
    b, s, d = x_bd.shape
    n = b * s
    out = _sc_kernel(x_bd.reshape(n, d))
    return out.reshape(b, s, d)
